# Initial kernel scaffold; baseline (speedup 1.0000x reference)
import jax
import jax.numpy as jnp
from jax.experimental import pallas as pl


def kernel(x, w_i2s_masked, b_i2s, w_s2s, b_s2s):
    raise NotImplementedError("write your pallas kernel here")


# speedup vs baseline: 2.2593x; 2.2593x over previous
import functools

import jax
import jax.numpy as jnp
import numpy as np
from jax.experimental import pallas as pl
from jax.experimental.pallas import tpu as pltpu


def _gate_permutation(out_features):
    O = 4 * out_features
    G = O // 3
    g4 = out_features // 3
    return np.asarray([clr * G + j * g4 + t
                       for j in range(4) for clr in range(3) for t in range(g4)])


def _rows_per_block(H, max_rows):
    for r in range(min(H, max_rows), 0, -1):
        if H % r == 0:
            return r
    return 1


def _make_body(ROWS, Bt, W, C, Fp, Op):
    M = Bt * W

    def body(xp_ref, wi_ref, ws_ref, b_ref, out_ref, i2s_ref, hpad_ref, c_ref):
        @pl.when(pl.program_id(1) == 0)
        def _init():
            hpad_ref[...] = jnp.zeros_like(hpad_ref)
            c_ref[...] = jnp.zeros_like(c_ref)

        xblk = xp_ref[...]
        xcat = jnp.concatenate(
            [xblk[:, :, 0:W, :], xblk[:, :, 1:W + 1, :]],
            axis=3).reshape(ROWS * M, 2 * C)
        i2s_ref[...] = (
            jnp.dot(xcat, wi_ref[...], preferred_element_type=jnp.float32)
            + b_ref[...]).reshape(ROWS, M, Op)

        ws = ws_ref[...]

        for r in range(ROWS):
            hpad = hpad_ref[...]
            hcat = jnp.concatenate(
                [hpad[:, 0:W, :], hpad[:, 1:W + 1, :], hpad[:, 2:W + 2, :]],
                axis=2).reshape(M, 3 * Fp)
            gates = i2s_ref[r] + jnp.dot(
                hcat, ws, preferred_element_type=jnp.float32)
            s = jax.nn.sigmoid(gates)
            f_g = s[:, 0 * Fp:1 * Fp]
            i_g = s[:, 1 * Fp:2 * Fp]
            o_g = s[:, 2 * Fp:3 * Fp]
            g_g = 2.0 * s[:, 3 * Fp:4 * Fp] - 1.0
            c_new = f_g * c_ref[...] + i_g * g_g
            c_ref[...] = c_new
            h_new = o_g * jnp.tanh(c_new)
            hpad_ref[:, 1:W + 1, :] = h_new.reshape(Bt, W, Fp).astype(jnp.bfloat16)
            out_ref[r] = h_new

    return body


def kernel(x, w_i2s_masked, b_i2s, w_s2s, b_s2s):
    B, C, H, W = x.shape
    F = w_s2s.shape[1]
    O = 4 * F
    Fp = ((F + 127) // 128) * 128
    Op = 4 * Fp
    BT = 2 if B % 2 == 0 else 1
    Bt = B // BT
    ROWS = _rows_per_block(H, 8)

    perm = _gate_permutation(F)

    wi = jnp.transpose(w_i2s_masked[:, :, 0, 0:2], (2, 1, 0)).reshape(2 * C, O)
    wi = wi[:, perm].reshape(2 * C, 4, F)
    wi = jnp.pad(wi, ((0, 0), (0, 0), (0, Fp - F))).reshape(2 * C, Op)
    ws = jnp.transpose(w_s2s, (2, 1, 0)).reshape(3 * F, O)
    ws = ws[:, perm].reshape(3, F, 4, F)
    ws = jnp.pad(ws, ((0, 0), (0, Fp - F), (0, 0), (0, Fp - F)))
    ws = ws.reshape(3 * Fp, Op)
    bias = (b_i2s + b_s2s)[perm].reshape(4, F)
    bias = jnp.pad(bias, ((0, 0), (0, Fp - F))).reshape(1, Op)

    gsc = jnp.concatenate([jnp.ones((3 * Fp,), jnp.float32),
                           jnp.full((Fp,), 2.0, jnp.float32)])
    wi = (wi * gsc).astype(jnp.bfloat16)
    ws = (ws * gsc).astype(jnp.bfloat16)
    bias = (bias * gsc).astype(jnp.float32)

    xt = jnp.transpose(x, (2, 0, 3, 1)).astype(jnp.bfloat16)
    xp = jnp.pad(xt, ((0, 0), (0, 0), (1, 0), (0, 0)))

    grid = (BT, H // ROWS)
    body = _make_body(ROWS, Bt, W, C, Fp, Op)

    out = pl.pallas_call(
        body,
        out_shape=jax.ShapeDtypeStruct((H, B * W, Fp), jnp.float32),
        grid_spec=pltpu.PrefetchScalarGridSpec(
            num_scalar_prefetch=0,
            grid=grid,
            in_specs=[
                pl.BlockSpec((ROWS, Bt, W + 1, C), lambda bt, rb: (rb, bt, 0, 0)),
                pl.BlockSpec((2 * C, Op), lambda bt, rb: (0, 0)),
                pl.BlockSpec((3 * Fp, Op), lambda bt, rb: (0, 0)),
                pl.BlockSpec((1, Op), lambda bt, rb: (0, 0)),
            ],
            out_specs=pl.BlockSpec((ROWS, Bt * W, Fp), lambda bt, rb: (rb, bt, 0)),
            scratch_shapes=[
                pltpu.VMEM((ROWS, Bt * W, Op), jnp.float32),
                pltpu.VMEM((Bt, W + 2, Fp), jnp.bfloat16),
                pltpu.VMEM((Bt * W, Fp), jnp.float32),
            ],
        ),
        compiler_params=pltpu.CompilerParams(
            dimension_semantics=("parallel", "arbitrary")),
    )(xp, wi, ws, bias)

    return jnp.transpose(out.reshape(H, B, W, Fp)[..., :F], (1, 3, 0, 2))
